# baseline (speedup 1.0000x reference)
import functools

import jax
import jax.numpy as jnp
from jax import lax
from jax.experimental import pallas as pl
from jax.experimental.pallas import tpu as pltpu
from jax.experimental.pallas import tpu_sc as plsc

B = 16384
D = 64
H1 = 128
H2 = 64

NC = 2
NS = 16
NW = NC * NS
B_PER_W = B // NW
CHUNK = 128
N_CHUNKS = B_PER_W // CHUNK


@functools.lru_cache(maxsize=None)
def _build_sc_gather():
    mesh = plsc.VectorSubcoreMesh(core_axis_name="c", subcore_axis_name="s")

    @functools.partial(
        pl.kernel,
        mesh=mesh,
        compiler_params=pltpu.CompilerParams(use_tc_tiling_on_sc=False),
        out_type=[
            jax.ShapeDtypeStruct((B, D), jnp.float32),
            jax.ShapeDtypeStruct((B, D), jnp.float32),
            jax.ShapeDtypeStruct((B,), jnp.float32),
            jax.ShapeDtypeStruct((B,), jnp.float32),
        ],
        scratch_types=[
            pltpu.VMEM((N_CHUNKS, CHUNK), jnp.int32),
            pltpu.VMEM((N_CHUNKS, CHUNK), jnp.int32),
            pltpu.VMEM((B_PER_W, D), jnp.float32),
            pltpu.VMEM((B_PER_W, D), jnp.float32),
            pltpu.VMEM((B_PER_W,), jnp.float32),
            pltpu.VMEM((B_PER_W,), jnp.float32),
            pltpu.SemaphoreType.DMA,
            pltpu.SemaphoreType.DMA,
            pltpu.SemaphoreType.DMA,
            pltpu.SemaphoreType.DMA,
        ],
    )
    def sc_gather(uid_hbm, iid_hbm, uet_hbm, iet_hbm, ubt_hbm, ibt_hbm,
                  ue_out, ie_out, ub_out, ib_out,
                  uidx_v, iidx_v, ue_v, ie_v, ub_v, ib_v,
                  sem_u, sem_i, sem_ub, sem_ib):
        wid = lax.axis_index("s") * NC + lax.axis_index("c")
        base = wid * B_PER_W
        row = wid * N_CHUNKS
        pltpu.sync_copy(uid_hbm.at[pl.ds(row, N_CHUNKS)], uidx_v)
        pltpu.sync_copy(iid_hbm.at[pl.ds(row, N_CHUNKS)], iidx_v)
        copies = []
        for j in range(N_CHUNKS):
            sl = pl.ds(j * CHUNK, CHUNK)
            copies.append(pltpu.async_copy(uet_hbm.at[uidx_v.at[j]], ue_v.at[sl], sem_u))
            copies.append(pltpu.async_copy(iet_hbm.at[iidx_v.at[j]], ie_v.at[sl], sem_i))
            copies.append(pltpu.async_copy(ubt_hbm.at[uidx_v.at[j]], ub_v.at[sl], sem_ub))
            copies.append(pltpu.async_copy(ibt_hbm.at[iidx_v.at[j]], ib_v.at[sl], sem_ib))
        for c in copies:
            c.wait()
        pltpu.sync_copy(ue_v, ue_out.at[pl.ds(base, B_PER_W)])
        pltpu.sync_copy(ie_v, ie_out.at[pl.ds(base, B_PER_W)])
        pltpu.sync_copy(ub_v, ub_out.at[pl.ds(base, B_PER_W)])
        pltpu.sync_copy(ib_v, ib_out.at[pl.ds(base, B_PER_W)])

    return sc_gather


BLK = 2048


def _mlp_body(ue_ref, ie_ref, ub_ref, ib_ref,
              w1u_ref, w1i_ref, b1_ref, w2_ref, b2_ref, w3_ref,
              b3_ref, gb_ref, out_ref):
    ue = ue_ref[...]
    ie = ie_ref[...]
    h = jnp.dot(ue, w1u_ref[...], preferred_element_type=jnp.float32)
    h = h + jnp.dot(ie, w1i_ref[...], preferred_element_type=jnp.float32)
    h = jnp.maximum(h + b1_ref[...], 0.0)
    h2 = jnp.dot(h, w2_ref[...], preferred_element_type=jnp.float32)
    h2 = jnp.maximum(h2 + b2_ref[...], 0.0)
    mlp = jnp.sum(h2 * w3_ref[...], axis=1)
    mf = jnp.sum(ue * ie, axis=1)
    out_ref[...] = mf + mlp + ub_ref[...] + ib_ref[...] + b3_ref[0, 0] + gb_ref[0, 0]


def _mlp_call(ue, ie, ub, ib, w1u, w1i, b1, w2, b2, w3, b3, gb):
    grid = B // BLK
    blk2 = lambda i: (i, 0)
    fix2 = lambda i: (0, 0)
    return pl.pallas_call(
        _mlp_body,
        grid=(grid,),
        in_specs=[
            pl.BlockSpec((BLK, D), blk2),
            pl.BlockSpec((BLK, D), blk2),
            pl.BlockSpec((BLK,), lambda i: (i,)),
            pl.BlockSpec((BLK,), lambda i: (i,)),
            pl.BlockSpec((D, H1), fix2),
            pl.BlockSpec((D, H1), fix2),
            pl.BlockSpec((1, H1), fix2),
            pl.BlockSpec((H1, H2), fix2),
            pl.BlockSpec((1, H2), fix2),
            pl.BlockSpec((1, H2), fix2),
            pl.BlockSpec((1, 1), fix2),
            pl.BlockSpec((1, 1), fix2),
        ],
        out_specs=pl.BlockSpec((BLK,), lambda i: (i,)),
        out_shape=jax.ShapeDtypeStruct((B,), jnp.float32),
    )(ue, ie, ub, ib, w1u, w1i, b1, w2, b2, w3, b3, gb)


def kernel(user_ids, item_ids, user_emb_table, item_emb_table,
           user_bias_table, item_bias_table, global_bias,
           W1, b1, W2, b2, W3, b3):
    uid2 = user_ids.astype(jnp.int32).reshape(B // CHUNK, CHUNK)
    iid2 = item_ids.astype(jnp.int32).reshape(B // CHUNK, CHUNK)
    ubt = user_bias_table.reshape(-1)
    ibt = item_bias_table.reshape(-1)
    ue, ie, ub, ib = _build_sc_gather()(uid2, iid2, user_emb_table, item_emb_table, ubt, ibt)
    return _mlp_call(
        ue, ie, ub, ib,
        W1[:D], W1[D:],
        b1.reshape(1, H1), W2, b2.reshape(1, H2), W3.reshape(1, H2),
        b3.reshape(1, 1), global_bias.reshape(1, 1),
    )


# speedup vs baseline: 1.2243x; 1.2243x over previous
import functools

import jax
import jax.numpy as jnp
from jax import lax
from jax.experimental import pallas as pl
from jax.experimental.pallas import tpu as pltpu
from jax.experimental.pallas import tpu_sc as plsc

B = 16384
D = 64
H1 = 128
H2 = 64
V = 1000000
VP = V // 2

NC = 2
NS = 16
NW = NC * NS
BPW = B // NW
CH = 128
NJ = BPW // CH
NR = 2

CW = 8192
RG = (V + CW - 1) // CW


def _repack_body(t_ref, out_ref):
    x = t_ref[...]
    xt = jnp.transpose(x, (1, 0))
    x3 = xt.reshape(CW // 2, 2, D)
    out_ref[...] = jnp.concatenate([x3[:, 0, :], x3[:, 1, :]], axis=1)


def _repack(tT):
    return pl.pallas_call(
        _repack_body,
        grid=(RG,),
        in_specs=[pl.BlockSpec((D, CW), lambda i: (0, i))],
        out_specs=pl.BlockSpec((CW // 2, 128), lambda i: (i, 0)),
        out_shape=jax.ShapeDtypeStruct((VP, 128), jnp.float32),
    )(tT)


@functools.lru_cache(maxsize=None)
def _build_sc_gather():
    mesh = plsc.VectorSubcoreMesh(core_axis_name="c", subcore_axis_name="s")

    @functools.partial(
        pl.kernel,
        mesh=mesh,
        out_type=[
            jax.ShapeDtypeStruct((B, 128), jnp.float32),
            jax.ShapeDtypeStruct((B, 128), jnp.float32),
        ],
        scratch_types=[
            pltpu.VMEM((BPW,), jnp.int32),
            pltpu.VMEM((BPW,), jnp.int32),
            pltpu.VMEM((BPW,), jnp.int32),
            pltpu.VMEM((BPW,), jnp.int32),
            pltpu.VMEM((NR * CH, 128), jnp.float32),
            pltpu.VMEM((NR * CH, 128), jnp.float32),
            pltpu.SemaphoreType.DMA,
            pltpu.SemaphoreType.DMA,
        ],
    )
    def sc_gather(uid_hbm, iid_hbm, upk_hbm, ipk_hbm,
                  ue_out, ie_out,
                  uid_v, iid_v, uq_v, iq_v, ustg_v, istg_v,
                  sem_u, sem_i):
        wid = lax.axis_index("s") * NC + lax.axis_index("c")
        base = wid * BPW
        pltpu.sync_copy(uid_hbm.at[pl.ds(base, BPW)], uid_v)
        pltpu.sync_copy(iid_hbm.at[pl.ds(base, BPW)], iid_v)

        def q_body(g, _):
            sl = pl.ds(g * 16, 16)
            uq_v[sl] = uid_v[sl] >> 1
            iq_v[sl] = iid_v[sl] >> 1
            return ()
        lax.fori_loop(0, BPW // 16, q_body, (), unroll=4)

        for r in range(NJ // NR):
            for p in range(NR):
                c = r * NR + p
                isl = pl.ds(c * CH, CH)
                dsl = pl.ds(p * CH, CH)
                pltpu.async_copy(upk_hbm.at[uq_v.at[isl]], ustg_v.at[dsl], sem_u)
                pltpu.async_copy(ipk_hbm.at[iq_v.at[isl]], istg_v.at[dsl], sem_i)
            pltpu.make_async_copy(
                upk_hbm.at[pl.ds(0, NR * CH)], ustg_v, sem_u).wait()
            pltpu.make_async_copy(
                ipk_hbm.at[pl.ds(0, NR * CH)], istg_v, sem_i).wait()
            osl = pl.ds(base + r * NR * CH, NR * CH)
            pltpu.sync_copy(ustg_v, ue_out.at[osl])
            pltpu.sync_copy(istg_v, ie_out.at[osl])

    return sc_gather


@functools.lru_cache(maxsize=None)
def _build_sc_bias():
    mesh = plsc.VectorSubcoreMesh(core_axis_name="c", subcore_axis_name="s")

    @functools.partial(
        pl.kernel,
        mesh=mesh,
        compiler_params=pltpu.CompilerParams(use_tc_tiling_on_sc=False),
        out_type=[
            jax.ShapeDtypeStruct((B,), jnp.float32),
            jax.ShapeDtypeStruct((B,), jnp.float32),
        ],
        scratch_types=[
            pltpu.VMEM((BPW,), jnp.int32),
            pltpu.VMEM((BPW,), jnp.int32),
            pltpu.VMEM((BPW,), jnp.float32),
            pltpu.VMEM((BPW,), jnp.float32),
            pltpu.SemaphoreType.DMA,
            pltpu.SemaphoreType.DMA,
        ],
    )
    def sc_bias(uid_hbm, iid_hbm, ubt_hbm, ibt_hbm,
                ub_out, ib_out,
                uid_v, iid_v, ub_v, ib_v, sem_ub, sem_ib):
        wid = lax.axis_index("s") * NC + lax.axis_index("c")
        base = wid * BPW
        pltpu.sync_copy(uid_hbm.at[pl.ds(base, BPW)], uid_v)
        pltpu.sync_copy(iid_hbm.at[pl.ds(base, BPW)], iid_v)
        for j in range(NJ):
            sl = pl.ds(j * CH, CH)
            pltpu.async_copy(ubt_hbm.at[uid_v.at[sl]], ub_v.at[sl], sem_ub)
            pltpu.async_copy(ibt_hbm.at[iid_v.at[sl]], ib_v.at[sl], sem_ib)
        pltpu.make_async_copy(ubt_hbm.at[pl.ds(0, BPW)], ub_v, sem_ub).wait()
        pltpu.make_async_copy(ibt_hbm.at[pl.ds(0, BPW)], ib_v, sem_ib).wait()
        pltpu.sync_copy(ub_v, ub_out.at[pl.ds(base, BPW)])
        pltpu.sync_copy(ib_v, ib_out.at[pl.ds(base, BPW)])

    return sc_bias


BLK = 2048


def _mlp_body(ue_ref, ie_ref, uid_ref, iid_ref, ub_ref, ib_ref,
              w1_ref, b1_ref, w2_ref, b2_ref, w3_ref, b3_ref, gb_ref,
              out_ref):
    ue2 = ue_ref[...]
    ie2 = ie_ref[...]
    uodd = (uid_ref[...] & 1)[:, None] == 1
    iodd = (iid_ref[...] & 1)[:, None] == 1
    ue = jnp.where(uodd, ue2[:, D:], ue2[:, :D])
    ie = jnp.where(iodd, ie2[:, D:], ie2[:, :D])
    w1 = w1_ref[...]
    h = jnp.dot(ue, w1[:D], preferred_element_type=jnp.float32)
    h = h + jnp.dot(ie, w1[D:], preferred_element_type=jnp.float32)
    h = jnp.maximum(h + b1_ref[...], 0.0)
    h2 = jnp.dot(h, w2_ref[...], preferred_element_type=jnp.float32)
    h2 = jnp.maximum(h2 + b2_ref[...], 0.0)
    mlp = jnp.sum(h2 * w3_ref[...], axis=1)
    mf = jnp.sum(ue * ie, axis=1)
    out_ref[...] = (mf + mlp + ub_ref[...] + ib_ref[...]
                    + b3_ref[0, 0] + gb_ref[0, 0])


def _mlp_call(ue2, ie2, uid, iid, ub, ib, w1, b1, w2, b2, w3, b3, gb):
    blk2 = lambda i: (i, 0)
    blk1 = lambda i: (i,)
    fix2 = lambda i: (0, 0)
    return pl.pallas_call(
        _mlp_body,
        grid=(B // BLK,),
        in_specs=[
            pl.BlockSpec((BLK, 128), blk2),
            pl.BlockSpec((BLK, 128), blk2),
            pl.BlockSpec((BLK,), blk1),
            pl.BlockSpec((BLK,), blk1),
            pl.BlockSpec((BLK,), blk1),
            pl.BlockSpec((BLK,), blk1),
            pl.BlockSpec((2 * D, H1), fix2),
            pl.BlockSpec((1, H1), fix2),
            pl.BlockSpec((H1, H2), fix2),
            pl.BlockSpec((1, H2), fix2),
            pl.BlockSpec((1, H2), fix2),
            pl.BlockSpec((1, 1), fix2),
            pl.BlockSpec((1, 1), fix2),
        ],
        out_specs=pl.BlockSpec((BLK,), blk1),
        out_shape=jax.ShapeDtypeStruct((B,), jnp.float32),
    )(ue2, ie2, uid, iid, ub, ib, w1, b1, w2, b2, w3, b3, gb)


def kernel(user_ids, item_ids, user_emb_table, item_emb_table,
           user_bias_table, item_bias_table, global_bias,
           W1, b1, W2, b2, W3, b3):
    uid = user_ids.astype(jnp.int32)
    iid = item_ids.astype(jnp.int32)
    upk = _repack(user_emb_table.T)
    ipk = _repack(item_emb_table.T)
    ue2, ie2 = _build_sc_gather()(uid, iid, upk, ipk)
    ub, ib = _build_sc_bias()(
        uid, iid, user_bias_table.reshape(V), item_bias_table.reshape(V))
    return _mlp_call(
        ue2, ie2, uid, iid, ub, ib,
        W1, b1.reshape(1, H1), W2, b2.reshape(1, H2), W3.reshape(1, H2),
        b3.reshape(1, 1), global_bias.reshape(1, 1),
    )


# speedup vs baseline: 1.7785x; 1.4527x over previous
import functools

import jax
import jax.numpy as jnp
from jax import lax
from jax.experimental import pallas as pl
from jax.experimental.pallas import tpu as pltpu
from jax.experimental.pallas import tpu_sc as plsc

B = 16384
D = 64
H1 = 128
H2 = 64
V = 1000000
VO = 1024000

NC = 2
NS = 16
NW = NC * NS
BPW = B // NW
CH = 128
NJ = BPW // CH
NR = 2

CW = 4096
RG = VO // CW


def _repack_body(ut_ref, it_ref, out_ref):
    out_ref[:, :D] = jnp.transpose(ut_ref[...], (1, 0))
    out_ref[:, D:] = jnp.transpose(it_ref[...], (1, 0))


def _repack(uT, iT):
    clamped = lambda i: (0, jnp.minimum(i, (V - 1) // CW))
    return pl.pallas_call(
        _repack_body,
        grid=(RG,),
        in_specs=[pl.BlockSpec((D, CW), clamped),
                  pl.BlockSpec((D, CW), clamped)],
        out_specs=pl.BlockSpec((CW, 128), lambda i: (i, 0)),
        out_shape=jax.ShapeDtypeStruct((VO, 128), jnp.float32),
    )(uT, iT)


@functools.lru_cache(maxsize=None)
def _build_sc_gather():
    mesh = plsc.VectorSubcoreMesh(core_axis_name="c", subcore_axis_name="s")

    @functools.partial(
        pl.kernel,
        mesh=mesh,
        out_type=[
            jax.ShapeDtypeStruct((B, 128), jnp.float32),
            jax.ShapeDtypeStruct((B, 128), jnp.float32),
        ],
        scratch_types=[
            pltpu.VMEM((BPW,), jnp.int32),
            pltpu.VMEM((BPW,), jnp.int32),
            pltpu.VMEM((BPW,), jnp.int32),
            pltpu.VMEM((BPW,), jnp.int32),
            pltpu.VMEM((NR * CH, 128), jnp.float32),
            pltpu.VMEM((NR * CH, 128), jnp.float32),
            pltpu.SemaphoreType.DMA,
            pltpu.SemaphoreType.DMA,
        ],
    )
    def sc_gather(uid_hbm, iid_hbm, pk_hbm,
                  ue_out, ie_out,
                  uid_v, iid_v, uq_v, iq_v, ustg_v, istg_v,
                  sem_u, sem_i):
        wid = lax.axis_index("s") * NC + lax.axis_index("c")
        base = wid * BPW
        pltpu.sync_copy(uid_hbm.at[pl.ds(base, BPW)], uid_v)
        pltpu.sync_copy(iid_hbm.at[pl.ds(base, BPW)], iid_v)

        for r in range(NJ // NR):
            for p in range(NR):
                c = r * NR + p
                isl = pl.ds(c * CH, CH)
                dsl = pl.ds(p * CH, CH)
                pltpu.async_copy(pk_hbm.at[uid_v.at[isl]], ustg_v.at[dsl], sem_u)
                pltpu.async_copy(pk_hbm.at[iid_v.at[isl]], istg_v.at[dsl], sem_i)
            pltpu.make_async_copy(
                pk_hbm.at[pl.ds(0, NR * CH)], ustg_v, sem_u).wait()
            pltpu.make_async_copy(
                pk_hbm.at[pl.ds(0, NR * CH)], istg_v, sem_i).wait()
            osl = pl.ds(base + r * NR * CH, NR * CH)
            pltpu.sync_copy(ustg_v, ue_out.at[osl])
            pltpu.sync_copy(istg_v, ie_out.at[osl])

    return sc_gather


@functools.lru_cache(maxsize=None)
def _build_sc_bias():
    mesh = plsc.VectorSubcoreMesh(core_axis_name="c", subcore_axis_name="s")

    @functools.partial(
        pl.kernel,
        mesh=mesh,
        compiler_params=pltpu.CompilerParams(use_tc_tiling_on_sc=False),
        out_type=[
            jax.ShapeDtypeStruct((B,), jnp.float32),
            jax.ShapeDtypeStruct((B,), jnp.float32),
        ],
        scratch_types=[
            pltpu.VMEM((BPW,), jnp.int32),
            pltpu.VMEM((BPW,), jnp.int32),
            pltpu.VMEM((BPW,), jnp.float32),
            pltpu.VMEM((BPW,), jnp.float32),
            pltpu.SemaphoreType.DMA,
            pltpu.SemaphoreType.DMA,
        ],
    )
    def sc_bias(uid_hbm, iid_hbm, ubt_hbm, ibt_hbm,
                ub_out, ib_out,
                uid_v, iid_v, ub_v, ib_v, sem_ub, sem_ib):
        wid = lax.axis_index("s") * NC + lax.axis_index("c")
        base = wid * BPW
        pltpu.sync_copy(uid_hbm.at[pl.ds(base, BPW)], uid_v)
        pltpu.sync_copy(iid_hbm.at[pl.ds(base, BPW)], iid_v)
        for j in range(NJ):
            sl = pl.ds(j * CH, CH)
            pltpu.async_copy(ubt_hbm.at[uid_v.at[sl]], ub_v.at[sl], sem_ub)
            pltpu.async_copy(ibt_hbm.at[iid_v.at[sl]], ib_v.at[sl], sem_ib)
        pltpu.make_async_copy(ubt_hbm.at[pl.ds(0, BPW)], ub_v, sem_ub).wait()
        pltpu.make_async_copy(ibt_hbm.at[pl.ds(0, BPW)], ib_v, sem_ib).wait()
        pltpu.sync_copy(ub_v, ub_out.at[pl.ds(base, BPW)])
        pltpu.sync_copy(ib_v, ib_out.at[pl.ds(base, BPW)])

    return sc_bias


BLK = 2048


def _mlp_body(ue_ref, ie_ref, ub_ref, ib_ref,
              w1_ref, b1_ref, w2_ref, b2_ref, w3_ref, b3_ref, gb_ref,
              out_ref):
    ue = ue_ref[...][:, :D]
    ie = ie_ref[...][:, D:]
    w1 = w1_ref[...]
    h = jnp.dot(ue, w1[:D], preferred_element_type=jnp.float32)
    h = h + jnp.dot(ie, w1[D:], preferred_element_type=jnp.float32)
    h = jnp.maximum(h + b1_ref[...], 0.0)
    h2 = jnp.dot(h, w2_ref[...], preferred_element_type=jnp.float32)
    h2 = jnp.maximum(h2 + b2_ref[...], 0.0)
    mlp = jnp.sum(h2 * w3_ref[...], axis=1)
    mf = jnp.sum(ue * ie, axis=1)
    out_ref[...] = (mf + mlp + ub_ref[...] + ib_ref[...]
                    + b3_ref[0, 0] + gb_ref[0, 0])


def _mlp_call(ue2, ie2, ub, ib, w1, b1, w2, b2, w3, b3, gb):
    blk2 = lambda i: (i, 0)
    blk1 = lambda i: (i,)
    fix2 = lambda i: (0, 0)
    return pl.pallas_call(
        _mlp_body,
        grid=(B // BLK,),
        in_specs=[
            pl.BlockSpec((BLK, 128), blk2),
            pl.BlockSpec((BLK, 128), blk2),
            pl.BlockSpec((BLK,), blk1),
            pl.BlockSpec((BLK,), blk1),
            pl.BlockSpec((2 * D, H1), fix2),
            pl.BlockSpec((1, H1), fix2),
            pl.BlockSpec((H1, H2), fix2),
            pl.BlockSpec((1, H2), fix2),
            pl.BlockSpec((1, H2), fix2),
            pl.BlockSpec((1, 1), fix2),
            pl.BlockSpec((1, 1), fix2),
        ],
        out_specs=pl.BlockSpec((BLK,), blk1),
        out_shape=jax.ShapeDtypeStruct((B,), jnp.float32),
    )(ue2, ie2, ub, ib, w1, b1, w2, b2, w3, b3, gb)


def kernel(user_ids, item_ids, user_emb_table, item_emb_table,
           user_bias_table, item_bias_table, global_bias,
           W1, b1, W2, b2, W3, b3):
    uid = user_ids.astype(jnp.int32)
    iid = item_ids.astype(jnp.int32)
    pk = _repack(user_emb_table.T, item_emb_table.T)
    ue2, ie2 = _build_sc_gather()(uid, iid, pk)
    ub, ib = _build_sc_bias()(
        uid, iid, user_bias_table.reshape(V), item_bias_table.reshape(V))
    return _mlp_call(
        ue2, ie2, ub, ib,
        W1, b1.reshape(1, H1), W2, b2.reshape(1, H2), W3.reshape(1, H2),
        b3.reshape(1, 1), global_bias.reshape(1, 1),
    )


# speedup vs baseline: 2.2117x; 1.2435x over previous
import functools

import jax
import jax.numpy as jnp
from jax import lax
from jax.experimental import pallas as pl
from jax.experimental.pallas import tpu as pltpu
from jax.experimental.pallas import tpu_sc as plsc

B = 16384
D = 64
H1 = 128
H2 = 64
V = 1000000
VO = 1024000

NC = 2
NS = 16
NW = NC * NS
BPW = B // NW
CH = 128
NJ = BPW // CH
NR = 2

CW = 4096
RG = VO // CW


def _repack_body(ut_ref, it_ref, out_ref):
    xs = jnp.concatenate([ut_ref[...], it_ref[...]], axis=0)
    out_ref[...] = jnp.transpose(xs, (1, 0))


def _repack(uT, iT):
    clamped = lambda i: (0, jnp.minimum(i, (V - 1) // CW))
    return pl.pallas_call(
        _repack_body,
        grid=(RG,),
        in_specs=[pl.BlockSpec((D, CW), clamped),
                  pl.BlockSpec((D, CW), clamped)],
        out_specs=pl.BlockSpec((CW, 128), lambda i: (i, 0)),
        out_shape=jax.ShapeDtypeStruct((VO, 128), jnp.float32),
    )(uT, iT)


@functools.lru_cache(maxsize=None)
def _build_sc_gather():
    mesh = plsc.VectorSubcoreMesh(core_axis_name="c", subcore_axis_name="s")

    @functools.partial(
        pl.kernel,
        mesh=mesh,
        out_type=[
            jax.ShapeDtypeStruct((B, 128), jnp.float32),
            jax.ShapeDtypeStruct((B, 128), jnp.float32),
        ],
        scratch_types=[
            pltpu.VMEM((BPW,), jnp.int32),
            pltpu.VMEM((BPW,), jnp.int32),
            pltpu.VMEM((BPW,), jnp.int32),
            pltpu.VMEM((BPW,), jnp.int32),
            pltpu.VMEM((NR * CH, 128), jnp.float32),
            pltpu.VMEM((NR * CH, 128), jnp.float32),
            pltpu.SemaphoreType.DMA,
            pltpu.SemaphoreType.DMA,
        ],
    )
    def sc_gather(uid_hbm, iid_hbm, pk_hbm,
                  ue_out, ie_out,
                  uid_v, iid_v, uq_v, iq_v, ustg_v, istg_v,
                  sem_u, sem_i):
        wid = lax.axis_index("s") * NC + lax.axis_index("c")
        base = wid * BPW
        pltpu.sync_copy(uid_hbm.at[pl.ds(base, BPW)], uid_v)
        pltpu.sync_copy(iid_hbm.at[pl.ds(base, BPW)], iid_v)

        for r in range(NJ // NR):
            for p in range(NR):
                c = r * NR + p
                isl = pl.ds(c * CH, CH)
                dsl = pl.ds(p * CH, CH)
                pltpu.async_copy(pk_hbm.at[uid_v.at[isl]], ustg_v.at[dsl], sem_u)
                pltpu.async_copy(pk_hbm.at[iid_v.at[isl]], istg_v.at[dsl], sem_i)
            pltpu.make_async_copy(
                pk_hbm.at[pl.ds(0, NR * CH)], ustg_v, sem_u).wait()
            pltpu.make_async_copy(
                pk_hbm.at[pl.ds(0, NR * CH)], istg_v, sem_i).wait()
            osl = pl.ds(base + r * NR * CH, NR * CH)
            pltpu.sync_copy(ustg_v, ue_out.at[osl])
            pltpu.sync_copy(istg_v, ie_out.at[osl])

    return sc_gather


@functools.lru_cache(maxsize=None)
def _build_sc_bias():
    mesh = plsc.VectorSubcoreMesh(core_axis_name="c", subcore_axis_name="s")

    @functools.partial(
        pl.kernel,
        mesh=mesh,
        compiler_params=pltpu.CompilerParams(use_tc_tiling_on_sc=False),
        out_type=[
            jax.ShapeDtypeStruct((B,), jnp.float32),
            jax.ShapeDtypeStruct((B,), jnp.float32),
        ],
        scratch_types=[
            pltpu.VMEM((BPW,), jnp.int32),
            pltpu.VMEM((BPW,), jnp.int32),
            pltpu.VMEM((BPW,), jnp.float32),
            pltpu.VMEM((BPW,), jnp.float32),
            pltpu.SemaphoreType.DMA,
            pltpu.SemaphoreType.DMA,
        ],
    )
    def sc_bias(uid_hbm, iid_hbm, ubt_hbm, ibt_hbm,
                ub_out, ib_out,
                uid_v, iid_v, ub_v, ib_v, sem_ub, sem_ib):
        wid = lax.axis_index("s") * NC + lax.axis_index("c")
        base = wid * BPW
        pltpu.sync_copy(uid_hbm.at[pl.ds(base, BPW)], uid_v)
        pltpu.sync_copy(iid_hbm.at[pl.ds(base, BPW)], iid_v)
        for j in range(NJ):
            sl = pl.ds(j * CH, CH)
            pltpu.async_copy(ubt_hbm.at[uid_v.at[sl]], ub_v.at[sl], sem_ub)
            pltpu.async_copy(ibt_hbm.at[iid_v.at[sl]], ib_v.at[sl], sem_ib)
        pltpu.make_async_copy(ubt_hbm.at[pl.ds(0, BPW)], ub_v, sem_ub).wait()
        pltpu.make_async_copy(ibt_hbm.at[pl.ds(0, BPW)], ib_v, sem_ib).wait()
        pltpu.sync_copy(ub_v, ub_out.at[pl.ds(base, BPW)])
        pltpu.sync_copy(ib_v, ib_out.at[pl.ds(base, BPW)])

    return sc_bias


BLK = 2048


def _mlp_body(ue_ref, ie_ref, ub_ref, ib_ref,
              w1_ref, b1_ref, w2_ref, b2_ref, w3_ref, b3_ref, gb_ref,
              out_ref):
    ue = ue_ref[...][:, :D]
    ie = ie_ref[...][:, D:]
    w1 = w1_ref[...]
    h = jnp.dot(ue, w1[:D], preferred_element_type=jnp.float32)
    h = h + jnp.dot(ie, w1[D:], preferred_element_type=jnp.float32)
    h = jnp.maximum(h + b1_ref[...], 0.0)
    h2 = jnp.dot(h, w2_ref[...], preferred_element_type=jnp.float32)
    h2 = jnp.maximum(h2 + b2_ref[...], 0.0)
    mlp = jnp.sum(h2 * w3_ref[...], axis=1)
    mf = jnp.sum(ue * ie, axis=1)
    out_ref[...] = (mf + mlp + ub_ref[...] + ib_ref[...]
                    + b3_ref[0, 0] + gb_ref[0, 0])


def _mlp_call(ue2, ie2, ub, ib, w1, b1, w2, b2, w3, b3, gb):
    blk2 = lambda i: (i, 0)
    blk1 = lambda i: (i,)
    fix2 = lambda i: (0, 0)
    return pl.pallas_call(
        _mlp_body,
        grid=(B // BLK,),
        in_specs=[
            pl.BlockSpec((BLK, 128), blk2),
            pl.BlockSpec((BLK, 128), blk2),
            pl.BlockSpec((BLK,), blk1),
            pl.BlockSpec((BLK,), blk1),
            pl.BlockSpec((2 * D, H1), fix2),
            pl.BlockSpec((1, H1), fix2),
            pl.BlockSpec((H1, H2), fix2),
            pl.BlockSpec((1, H2), fix2),
            pl.BlockSpec((1, H2), fix2),
            pl.BlockSpec((1, 1), fix2),
            pl.BlockSpec((1, 1), fix2),
        ],
        out_specs=pl.BlockSpec((BLK,), blk1),
        out_shape=jax.ShapeDtypeStruct((B,), jnp.float32),
    )(ue2, ie2, ub, ib, w1, b1, w2, b2, w3, b3, gb)


def kernel(user_ids, item_ids, user_emb_table, item_emb_table,
           user_bias_table, item_bias_table, global_bias,
           W1, b1, W2, b2, W3, b3):
    uid = user_ids.astype(jnp.int32)
    iid = item_ids.astype(jnp.int32)
    pk = _repack(user_emb_table.T, item_emb_table.T)
    ue2, ie2 = _build_sc_gather()(uid, iid, pk)
    ub, ib = _build_sc_bias()(
        uid, iid, user_bias_table.reshape(V), item_bias_table.reshape(V))
    return _mlp_call(
        ue2, ie2, ub, ib,
        W1, b1.reshape(1, H1), W2, b2.reshape(1, H2), W3.reshape(1, H2),
        b3.reshape(1, 1), global_bias.reshape(1, 1),
    )


# speedup vs baseline: 2.4909x; 1.1262x over previous
import functools

import jax
import jax.numpy as jnp
from jax import lax
from jax.experimental import pallas as pl
from jax.experimental.pallas import tpu as pltpu
from jax.experimental.pallas import tpu_sc as plsc

B = 16384
D = 64
H1 = 128
H2 = 64
V = 1000000
VO = 1024000

NC = 2
NS = 16
NW = NC * NS
BPW = B // NW
CH = 128
NJ = BPW // CH
NR = 2

CW = 8192
RG = VO // CW


def _repack_body(ut_ref, it_ref, out_ref):
    xs = jnp.concatenate([ut_ref[...], it_ref[...]], axis=0)
    out_ref[...] = jnp.transpose(xs, (1, 0))


def _repack(uT, iT):
    clamped = lambda i: (0, jnp.minimum(i, (V - 1) // CW))
    return pl.pallas_call(
        _repack_body,
        grid=(RG,),
        in_specs=[pl.BlockSpec((D, CW), clamped),
                  pl.BlockSpec((D, CW), clamped)],
        out_specs=pl.BlockSpec((CW, 128), lambda i: (i, 0)),
        out_shape=jax.ShapeDtypeStruct((VO, 128), jnp.float32),
    )(uT, iT)


@functools.lru_cache(maxsize=None)
def _build_sc_gather():
    mesh = plsc.VectorSubcoreMesh(core_axis_name="c", subcore_axis_name="s")

    @functools.partial(
        pl.kernel,
        mesh=mesh,
        out_type=[
            jax.ShapeDtypeStruct((B, 128), jnp.float32),
            jax.ShapeDtypeStruct((B, 128), jnp.float32),
        ],
        scratch_types=[
            pltpu.VMEM((BPW,), jnp.int32),
            pltpu.VMEM((BPW,), jnp.int32),
            pltpu.VMEM((BPW,), jnp.int32),
            pltpu.VMEM((BPW,), jnp.int32),
            pltpu.VMEM((NR * CH, 128), jnp.float32),
            pltpu.VMEM((NR * CH, 128), jnp.float32),
            pltpu.SemaphoreType.DMA,
            pltpu.SemaphoreType.DMA,
        ],
    )
    def sc_gather(uid_hbm, iid_hbm, pk_hbm,
                  ue_out, ie_out,
                  uid_v, iid_v, uq_v, iq_v, ustg_v, istg_v,
                  sem_u, sem_i):
        wid = lax.axis_index("s") * NC + lax.axis_index("c")
        base = wid * BPW
        pltpu.sync_copy(uid_hbm.at[pl.ds(base, BPW)], uid_v)
        pltpu.sync_copy(iid_hbm.at[pl.ds(base, BPW)], iid_v)

        for r in range(NJ // NR):
            for p in range(NR):
                c = r * NR + p
                isl = pl.ds(c * CH, CH)
                dsl = pl.ds(p * CH, CH)
                pltpu.async_copy(pk_hbm.at[uid_v.at[isl]], ustg_v.at[dsl], sem_u)
                pltpu.async_copy(pk_hbm.at[iid_v.at[isl]], istg_v.at[dsl], sem_i)
            pltpu.make_async_copy(
                pk_hbm.at[pl.ds(0, NR * CH)], ustg_v, sem_u).wait()
            pltpu.make_async_copy(
                pk_hbm.at[pl.ds(0, NR * CH)], istg_v, sem_i).wait()
            osl = pl.ds(base + r * NR * CH, NR * CH)
            pltpu.sync_copy(ustg_v, ue_out.at[osl])
            pltpu.sync_copy(istg_v, ie_out.at[osl])

    return sc_gather


@functools.lru_cache(maxsize=None)
def _build_sc_bias():
    mesh = plsc.VectorSubcoreMesh(core_axis_name="c", subcore_axis_name="s")

    @functools.partial(
        pl.kernel,
        mesh=mesh,
        compiler_params=pltpu.CompilerParams(use_tc_tiling_on_sc=False),
        out_type=[
            jax.ShapeDtypeStruct((B,), jnp.float32),
            jax.ShapeDtypeStruct((B,), jnp.float32),
        ],
        scratch_types=[
            pltpu.VMEM((BPW,), jnp.int32),
            pltpu.VMEM((BPW,), jnp.int32),
            pltpu.VMEM((BPW,), jnp.float32),
            pltpu.VMEM((BPW,), jnp.float32),
            pltpu.SemaphoreType.DMA,
            pltpu.SemaphoreType.DMA,
        ],
    )
    def sc_bias(uid_hbm, iid_hbm, ubt_hbm, ibt_hbm,
                ub_out, ib_out,
                uid_v, iid_v, ub_v, ib_v, sem_ub, sem_ib):
        wid = lax.axis_index("s") * NC + lax.axis_index("c")
        base = wid * BPW
        pltpu.sync_copy(uid_hbm.at[pl.ds(base, BPW)], uid_v)
        pltpu.sync_copy(iid_hbm.at[pl.ds(base, BPW)], iid_v)
        for j in range(NJ):
            sl = pl.ds(j * CH, CH)
            pltpu.async_copy(ubt_hbm.at[uid_v.at[sl]], ub_v.at[sl], sem_ub)
            pltpu.async_copy(ibt_hbm.at[iid_v.at[sl]], ib_v.at[sl], sem_ib)
        pltpu.make_async_copy(ubt_hbm.at[pl.ds(0, BPW)], ub_v, sem_ub).wait()
        pltpu.make_async_copy(ibt_hbm.at[pl.ds(0, BPW)], ib_v, sem_ib).wait()
        pltpu.sync_copy(ub_v, ub_out.at[pl.ds(base, BPW)])
        pltpu.sync_copy(ib_v, ib_out.at[pl.ds(base, BPW)])

    return sc_bias


BLK = 4096


def _mlp_body(ue_ref, ie_ref, ub_ref, ib_ref,
              w1_ref, b1_ref, w2_ref, b2_ref, w3_ref, b3_ref, gb_ref,
              out_ref):
    ue = ue_ref[...][:, :D]
    ie = ie_ref[...][:, D:]
    w1 = w1_ref[...]
    h = jnp.dot(ue, w1[:D], preferred_element_type=jnp.float32)
    h = h + jnp.dot(ie, w1[D:], preferred_element_type=jnp.float32)
    h = jnp.maximum(h + b1_ref[...], 0.0)
    h2 = jnp.dot(h, w2_ref[...], preferred_element_type=jnp.float32)
    h2 = jnp.maximum(h2 + b2_ref[...], 0.0)
    mlp = jnp.sum(h2 * w3_ref[...], axis=1)
    mf = jnp.sum(ue * ie, axis=1)
    out_ref[...] = (mf + mlp + ub_ref[...] + ib_ref[...]
                    + b3_ref[0, 0] + gb_ref[0, 0])


def _mlp_call(ue2, ie2, ub, ib, w1, b1, w2, b2, w3, b3, gb):
    blk2 = lambda i: (i, 0)
    blk1 = lambda i: (i,)
    fix2 = lambda i: (0, 0)
    return pl.pallas_call(
        _mlp_body,
        grid=(B // BLK,),
        in_specs=[
            pl.BlockSpec((BLK, 128), blk2),
            pl.BlockSpec((BLK, 128), blk2),
            pl.BlockSpec((BLK,), blk1),
            pl.BlockSpec((BLK,), blk1),
            pl.BlockSpec((2 * D, H1), fix2),
            pl.BlockSpec((1, H1), fix2),
            pl.BlockSpec((H1, H2), fix2),
            pl.BlockSpec((1, H2), fix2),
            pl.BlockSpec((1, H2), fix2),
            pl.BlockSpec((1, 1), fix2),
            pl.BlockSpec((1, 1), fix2),
        ],
        out_specs=pl.BlockSpec((BLK,), blk1),
        out_shape=jax.ShapeDtypeStruct((B,), jnp.float32),
    )(ue2, ie2, ub, ib, w1, b1, w2, b2, w3, b3, gb)


def kernel(user_ids, item_ids, user_emb_table, item_emb_table,
           user_bias_table, item_bias_table, global_bias,
           W1, b1, W2, b2, W3, b3):
    uid = user_ids.astype(jnp.int32)
    iid = item_ids.astype(jnp.int32)
    pk = _repack(user_emb_table.T, item_emb_table.T)
    ue2, ie2 = _build_sc_gather()(uid, iid, pk)
    ub, ib = _build_sc_bias()(
        uid, iid, user_bias_table.reshape(V), item_bias_table.reshape(V))
    return _mlp_call(
        ue2, ie2, ub, ib,
        W1, b1.reshape(1, H1), W2, b2.reshape(1, H2), W3.reshape(1, H2),
        b3.reshape(1, 1), global_bias.reshape(1, 1),
    )


# speedup vs baseline: 2.5031x; 1.0049x over previous
import functools

import jax
import jax.numpy as jnp
from jax import lax
from jax.experimental import pallas as pl
from jax.experimental.pallas import tpu as pltpu
from jax.experimental.pallas import tpu_sc as plsc

B = 16384
D = 64
H1 = 128
H2 = 64
V = 1000000
VO = 1048576

NC = 2
NS = 16
NW = NC * NS
BPW = B // NW
CH = 128
NJ = BPW // CH
NR = 2

CW = 16384
RG = VO // CW


def _repack_body(ut_ref, it_ref, out_ref):
    xs = jnp.concatenate([ut_ref[...], it_ref[...]], axis=0)
    out_ref[...] = jnp.transpose(xs, (1, 0))


def _repack(uT, iT):
    clamped = lambda i: (0, jnp.minimum(i, (V - 1) // CW))
    return pl.pallas_call(
        _repack_body,
        grid=(RG,),
        in_specs=[pl.BlockSpec((D, CW), clamped),
                  pl.BlockSpec((D, CW), clamped)],
        out_specs=pl.BlockSpec((CW, 128), lambda i: (i, 0)),
        out_shape=jax.ShapeDtypeStruct((VO, 128), jnp.float32),
    )(uT, iT)


@functools.lru_cache(maxsize=None)
def _build_sc_gather():
    mesh = plsc.VectorSubcoreMesh(core_axis_name="c", subcore_axis_name="s")

    @functools.partial(
        pl.kernel,
        mesh=mesh,
        out_type=[
            jax.ShapeDtypeStruct((B, 128), jnp.float32),
            jax.ShapeDtypeStruct((B, 128), jnp.float32),
        ],
        scratch_types=[
            pltpu.VMEM((BPW,), jnp.int32),
            pltpu.VMEM((BPW,), jnp.int32),
            pltpu.VMEM((BPW,), jnp.int32),
            pltpu.VMEM((BPW,), jnp.int32),
            pltpu.VMEM((NR * CH, 128), jnp.float32),
            pltpu.VMEM((NR * CH, 128), jnp.float32),
            pltpu.SemaphoreType.DMA,
            pltpu.SemaphoreType.DMA,
        ],
    )
    def sc_gather(uid_hbm, iid_hbm, pk_hbm,
                  ue_out, ie_out,
                  uid_v, iid_v, uq_v, iq_v, ustg_v, istg_v,
                  sem_u, sem_i):
        wid = lax.axis_index("s") * NC + lax.axis_index("c")
        base = wid * BPW
        pltpu.sync_copy(uid_hbm.at[pl.ds(base, BPW)], uid_v)
        pltpu.sync_copy(iid_hbm.at[pl.ds(base, BPW)], iid_v)

        for r in range(NJ // NR):
            for p in range(NR):
                c = r * NR + p
                isl = pl.ds(c * CH, CH)
                dsl = pl.ds(p * CH, CH)
                pltpu.async_copy(pk_hbm.at[uid_v.at[isl]], ustg_v.at[dsl], sem_u)
                pltpu.async_copy(pk_hbm.at[iid_v.at[isl]], istg_v.at[dsl], sem_i)
            pltpu.make_async_copy(
                pk_hbm.at[pl.ds(0, NR * CH)], ustg_v, sem_u).wait()
            pltpu.make_async_copy(
                pk_hbm.at[pl.ds(0, NR * CH)], istg_v, sem_i).wait()
            osl = pl.ds(base + r * NR * CH, NR * CH)
            pltpu.sync_copy(ustg_v, ue_out.at[osl])
            pltpu.sync_copy(istg_v, ie_out.at[osl])

    return sc_gather


@functools.lru_cache(maxsize=None)
def _build_sc_bias():
    mesh = plsc.VectorSubcoreMesh(core_axis_name="c", subcore_axis_name="s")

    @functools.partial(
        pl.kernel,
        mesh=mesh,
        compiler_params=pltpu.CompilerParams(use_tc_tiling_on_sc=False),
        out_type=[
            jax.ShapeDtypeStruct((B,), jnp.float32),
            jax.ShapeDtypeStruct((B,), jnp.float32),
        ],
        scratch_types=[
            pltpu.VMEM((BPW,), jnp.int32),
            pltpu.VMEM((BPW,), jnp.int32),
            pltpu.VMEM((BPW,), jnp.float32),
            pltpu.VMEM((BPW,), jnp.float32),
            pltpu.SemaphoreType.DMA,
            pltpu.SemaphoreType.DMA,
        ],
    )
    def sc_bias(uid_hbm, iid_hbm, ubt_hbm, ibt_hbm,
                ub_out, ib_out,
                uid_v, iid_v, ub_v, ib_v, sem_ub, sem_ib):
        wid = lax.axis_index("s") * NC + lax.axis_index("c")
        base = wid * BPW
        pltpu.sync_copy(uid_hbm.at[pl.ds(base, BPW)], uid_v)
        pltpu.sync_copy(iid_hbm.at[pl.ds(base, BPW)], iid_v)
        for j in range(NJ):
            sl = pl.ds(j * CH, CH)
            pltpu.async_copy(ubt_hbm.at[uid_v.at[sl]], ub_v.at[sl], sem_ub)
            pltpu.async_copy(ibt_hbm.at[iid_v.at[sl]], ib_v.at[sl], sem_ib)
        pltpu.make_async_copy(ubt_hbm.at[pl.ds(0, BPW)], ub_v, sem_ub).wait()
        pltpu.make_async_copy(ibt_hbm.at[pl.ds(0, BPW)], ib_v, sem_ib).wait()
        pltpu.sync_copy(ub_v, ub_out.at[pl.ds(base, BPW)])
        pltpu.sync_copy(ib_v, ib_out.at[pl.ds(base, BPW)])

    return sc_bias


BLK = 4096


def _mlp_body(ue_ref, ie_ref, ub_ref, ib_ref,
              w1_ref, b1_ref, w2_ref, b2_ref, w3_ref, b3_ref, gb_ref,
              out_ref):
    ue = ue_ref[...][:, :D]
    ie = ie_ref[...][:, D:]
    w1 = w1_ref[...]
    h = jnp.dot(ue, w1[:D], preferred_element_type=jnp.float32)
    h = h + jnp.dot(ie, w1[D:], preferred_element_type=jnp.float32)
    h = jnp.maximum(h + b1_ref[...], 0.0)
    h2 = jnp.dot(h, w2_ref[...], preferred_element_type=jnp.float32)
    h2 = jnp.maximum(h2 + b2_ref[...], 0.0)
    mlp = jnp.sum(h2 * w3_ref[...], axis=1)
    mf = jnp.sum(ue * ie, axis=1)
    out_ref[...] = (mf + mlp + ub_ref[...] + ib_ref[...]
                    + b3_ref[0, 0] + gb_ref[0, 0])


def _mlp_call(ue2, ie2, ub, ib, w1, b1, w2, b2, w3, b3, gb):
    blk2 = lambda i: (i, 0)
    blk1 = lambda i: (i,)
    fix2 = lambda i: (0, 0)
    return pl.pallas_call(
        _mlp_body,
        grid=(B // BLK,),
        in_specs=[
            pl.BlockSpec((BLK, 128), blk2),
            pl.BlockSpec((BLK, 128), blk2),
            pl.BlockSpec((BLK,), blk1),
            pl.BlockSpec((BLK,), blk1),
            pl.BlockSpec((2 * D, H1), fix2),
            pl.BlockSpec((1, H1), fix2),
            pl.BlockSpec((H1, H2), fix2),
            pl.BlockSpec((1, H2), fix2),
            pl.BlockSpec((1, H2), fix2),
            pl.BlockSpec((1, 1), fix2),
            pl.BlockSpec((1, 1), fix2),
        ],
        out_specs=pl.BlockSpec((BLK,), blk1),
        out_shape=jax.ShapeDtypeStruct((B,), jnp.float32),
    )(ue2, ie2, ub, ib, w1, b1, w2, b2, w3, b3, gb)


def kernel(user_ids, item_ids, user_emb_table, item_emb_table,
           user_bias_table, item_bias_table, global_bias,
           W1, b1, W2, b2, W3, b3):
    uid = user_ids.astype(jnp.int32)
    iid = item_ids.astype(jnp.int32)
    pk = _repack(user_emb_table.T, item_emb_table.T)
    ue2, ie2 = _build_sc_gather()(uid, iid, pk)
    ub, ib = _build_sc_bias()(
        uid, iid, user_bias_table.reshape(V), item_bias_table.reshape(V))
    return _mlp_call(
        ue2, ie2, ub, ib,
        W1, b1.reshape(1, H1), W2, b2.reshape(1, H2), W3.reshape(1, H2),
        b3.reshape(1, 1), global_bias.reshape(1, 1),
    )
